# baseline (speedup 1.0000x reference)
import functools

import jax
import jax.numpy as jnp
from jax.experimental import pallas as pl
from jax.experimental.pallas import tpu as pltpu

_NA = 3
_NC = 80
_C = _NC + 5
_AW = (10.0, 16.0, 33.0)
_AH = (13.0, 30.0, 23.0)


def _yolo_body(stride_ref, x_ref, o_ref, *, g):
    i = pl.program_id(0)
    a = jax.lax.rem(i, _NA)
    v = x_ref[0]
    s = jax.nn.sigmoid(v)
    e = jnp.exp(v)
    rid = jax.lax.broadcasted_iota(jnp.int32, v.shape, 0)
    cid = jax.lax.broadcasted_iota(jnp.int32, v.shape, 1)
    stride = stride_ref[0, 0]
    gx = jax.lax.rem(cid, g).astype(jnp.float32)
    gy = jax.lax.div(cid, g).astype(jnp.float32)
    grid_off = jnp.where(rid == 0, gx, gy)
    aw = jnp.where(a == 0, _AW[0], jnp.where(a == 1, _AW[1], _AW[2]))
    ah = jnp.where(a == 0, _AH[0], jnp.where(a == 1, _AH[1], _AH[2]))
    anch = jnp.where(rid == 2, aw, ah)
    box01 = (s + grid_off) * stride
    box23 = e * anch
    out = jnp.where(rid < 2, box01, jnp.where(rid < 4, box23, s))
    outp = jnp.concatenate(
        [out, jnp.zeros((128 - _C, g * g), jnp.float32)], axis=0)
    o_ref[0] = outp.T


def kernel(x, img_dim):
    B = x.shape[0]
    g = x.shape[2]
    hw = g * g
    n = B * _NA
    stride = (jnp.asarray(img_dim, jnp.float32) / g).reshape(1, 1)
    xv = x.reshape(n, _C, hw)
    out = pl.pallas_call(
        functools.partial(_yolo_body, g=g),
        grid=(n,),
        in_specs=[
            pl.BlockSpec(memory_space=pltpu.SMEM),
            pl.BlockSpec((1, _C, hw), lambda i: (i, 0, 0)),
        ],
        out_specs=pl.BlockSpec((1, hw, 128), lambda i: (i, 0, 0)),
        out_shape=jax.ShapeDtypeStruct((n, hw, 128), jnp.float32),
    )(stride, xv)
    return out[:, :, :_C].reshape(B, _NA * hw, _C)
